# baseline (speedup 1.0000x reference)
import functools

import jax
import jax.numpy as jnp
from jax import lax
from jax.experimental import pallas as pl
from jax.experimental.pallas import tpu as pltpu
from jax.experimental.pallas import tpu_sc as plsc

N_PAD = 10240
N_ACC = 10112
CHUNK = 96
NUM_SC = 2
NUM_SUB = 16
NW = NUM_SC * NUM_SUB
D = 128
ROW_BLK = 1280


PAY_W = 32


def _make_sc_scatter(e_pad, indirect):
    cpt = e_pad // (NW * CHUNK)
    rps = N_ACC // NUM_SUB
    mesh = plsc.VectorSubcoreMesh(core_axis_name="c", subcore_axis_name="s")

    NB = 4

    def body(*refs):
        if indirect:
            feat, src, dst, zd, p_out = refs[:5]
            rest = refs[5:]
            srcs, rest = rest[:NB], rest[NB:]
        else:
            feat, dst, zd, p_out = refs[:4]
            rest = refs[4:]
            srcs = (None,) * NB
            src = None
        dsts, rest = rest[:NB], rest[NB:]
        rows, rest = rest[:NB], rest[NB:]
        acc_sh, isem, gsem = rest

        c = lax.axis_index("c")
        s = lax.axis_index("s")
        wid = s * NUM_SC + c
        r0 = pl.multiple_of(s * rps, 8)
        tile_base = wid * (cpt * CHUNK)

        def hslice(ci):
            return pl.ds(pl.multiple_of(tile_base + ci * CHUNK, 8), CHUNK)

        def issue_idx(ci, b):
            pltpu.async_copy(dst.at[hslice(ci)], dsts[b], isem)
            if indirect:
                pltpu.async_copy(src.at[hslice(ci)], srcs[b], isem)

        def wait_idx(ci, b):
            pltpu.make_async_copy(dst.at[hslice(ci)], dsts[b], isem).wait()
            if indirect:
                pltpu.make_async_copy(src.at[hslice(ci)], srcs[b], isem).wait()

        def issue_gather(ci, b):
            if indirect:
                pltpu.async_copy(feat.at[srcs[b]], rows[b], gsem)
            else:
                pltpu.async_copy(feat.at[hslice(ci)], rows[b], gsem)

        def wait_gather(b):
            pltpu.make_async_copy(feat.at[pl.ds(0, CHUNK)], rows[b],
                                  gsem).wait()

        def step(ci, b, drain, gather_next, idx_next):
            if gather_next:
                wait_idx(ci + 2, (b + 2) % NB)
                issue_gather(ci + 2, (b + 2) % NB)
            if idx_next:
                issue_idx(ci + 3, (b + 3) % NB)
            wait_gather(b)
            pltpu.sync_copy(rows[b], acc_sh.at[dsts[b]], add=True)

        pltpu.sync_copy(zd, acc_sh.at[pl.ds(r0, rps)])
        plsc.subcore_barrier()

        issue_idx(0, 0)
        issue_idx(1, 1)
        issue_idx(2, 2)
        wait_idx(0, 0)
        issue_gather(0, 0)
        wait_idx(1, 1)
        issue_gather(1, 1)
        step(0, 0, False, True, True)
        step(1, 1, False, True, True)

        nq = max(0, (cpt - 6) // NB)

        def spin(q, carry):
            for j in range(NB):
                ci = 2 + NB * q + j
                step(ci, (2 + j) % NB, True, True, True)
            return carry

        lax.fori_loop(0, nq, spin, 0)
        for ci in range(2 + NB * nq, cpt):
            step(ci, ci % NB, True, ci + 2 < cpt, ci + 3 < cpt)
        plsc.subcore_barrier()

        pltpu.sync_copy(acc_sh.at[pl.ds(r0, rps)], p_out.at[c, pl.ds(r0, rps)])

    scratch = []
    if indirect:
        scratch += [pltpu.VMEM((CHUNK,), jnp.int32)] * NB
    scratch += [pltpu.VMEM((CHUNK,), jnp.int32)] * NB
    scratch += [pltpu.VMEM((CHUNK, D), jnp.float32)] * NB
    scratch += [
        pltpu.VMEM_SHARED((N_ACC, D), jnp.float32),
        pltpu.SemaphoreType.DMA,
        pltpu.SemaphoreType.DMA,
    ]
    return pl.kernel(
        body, out_type=[jax.ShapeDtypeStruct((NUM_SC, N_PAD, D), jnp.float32)],
        mesh=mesh, scratch_types=scratch)


def _tc_layer_body(x_ref, p_ref, ds_ref, wa_ref, wb_ref, we_ref, b_ref,
                   o_ref, *, relu):
    ds = ds_ref[0] + ds_ref[1]
    deg = ds[:, 4:5] + 1.0
    sh = x_ref[...] + p_ref[0] + p_ref[1]
    acc = jnp.dot(sh, wb_ref[...], preferred_element_type=jnp.float32)
    acc += deg * jnp.dot(x_ref[...], wa_ref[...],
                         preferred_element_type=jnp.float32)
    acc += jnp.dot(ds, we_ref[...], preferred_element_type=jnp.float32)
    acc += b_ref[...]
    o_ref[...] = jnp.maximum(acc, 0.0) if relu else acc


def _tc_layer(x, p, ds, wa, wb, we, b, relu):
    din, dout = wa.shape
    grid = N_PAD // ROW_BLK
    return pl.pallas_call(
        functools.partial(_tc_layer_body, relu=relu),
        grid=(grid,),
        in_specs=[
            pl.BlockSpec((ROW_BLK, din), lambda i: (i, 0)),
            pl.BlockSpec((NUM_SC, ROW_BLK, din), lambda i: (0, i, 0)),
            pl.BlockSpec((NUM_SC, ROW_BLK, D), lambda i: (0, i, 0)),
            pl.BlockSpec((din, dout), lambda i: (0, 0)),
            pl.BlockSpec((din, dout), lambda i: (0, 0)),
            pl.BlockSpec((D, dout), lambda i: (0, 0)),
            pl.BlockSpec((1, dout), lambda i: (0, 0)),
        ],
        out_specs=pl.BlockSpec((ROW_BLK, dout), lambda i: (i, 0)),
        out_shape=jax.ShapeDtypeStruct((N_PAD, dout), jnp.float32),
    )(x, p, ds, wa, wb, we, b)


def _tc_final_body(x_ref, p_ref, ds_ref, wa_ref, wb_ref, we_ref, b_ref,
                   wc_ref, bc_ref, o_ref):
    ds = ds_ref[0] + ds_ref[1]
    deg = ds[:, 4:5] + 1.0
    sh = x_ref[...] + p_ref[0] + p_ref[1]
    h = jnp.dot(sh, wb_ref[...], preferred_element_type=jnp.float32)
    h += deg * jnp.dot(x_ref[...], wa_ref[...],
                       preferred_element_type=jnp.float32)
    h += jnp.dot(ds, we_ref[...], preferred_element_type=jnp.float32)
    h += b_ref[...]
    logits = jnp.dot(h, wc_ref[...],
                     preferred_element_type=jnp.float32) + bc_ref[...]
    m = jnp.max(logits, axis=1, keepdims=True)
    lse = m + jnp.log(jnp.sum(jnp.exp(logits - m), axis=1, keepdims=True))
    o_ref[...] = logits - lse


def _tc_final(x, p, ds, wa, wb, we, b, wc, bc):
    din, dout = wa.shape
    nc = wc.shape[1]
    grid = N_PAD // ROW_BLK
    return pl.pallas_call(
        _tc_final_body,
        grid=(grid,),
        in_specs=[
            pl.BlockSpec((ROW_BLK, din), lambda i: (i, 0)),
            pl.BlockSpec((NUM_SC, ROW_BLK, din), lambda i: (0, i, 0)),
            pl.BlockSpec((NUM_SC, ROW_BLK, D), lambda i: (0, i, 0)),
            pl.BlockSpec((din, dout), lambda i: (0, 0)),
            pl.BlockSpec((din, dout), lambda i: (0, 0)),
            pl.BlockSpec((D, dout), lambda i: (0, 0)),
            pl.BlockSpec((1, dout), lambda i: (0, 0)),
            pl.BlockSpec((dout, nc), lambda i: (0, 0)),
            pl.BlockSpec((1, nc), lambda i: (0, 0)),
        ],
        out_specs=pl.BlockSpec((ROW_BLK, nc), lambda i: (i, 0)),
        out_shape=jax.ShapeDtypeStruct((N_PAD, nc), jnp.float32),
    )(x, p, ds, wa, wb, we, b, wc, bc)


def _split_w(w, din, b):
    de = w.shape[0] - 2 * din
    wa = w[:din]
    wb = w[din:2 * din]
    we = jnp.zeros((D, w.shape[1]), jnp.float32)
    we = we.at[:de].set(w[2 * din:])
    we = we.at[4].set(b)
    return wa, wb, we


def kernel(x, edge_index, edge_attr, W1, b1, W2, b2, W3, b3, Wc, bc):
    n = x.shape[0]
    e = edge_index.shape[1]
    cpt = -(-e // (NW * CHUNK))
    if cpt % 2 == 0:
        cpt += 1
    e_pad = cpt * NW * CHUNK

    src = jnp.pad(edge_index[0].astype(jnp.int32), (0, e_pad - e))
    dst = jnp.pad(edge_index[1].astype(jnp.int32), (0, e_pad - e),
                  constant_values=n)
    de = edge_attr.shape[1]
    pay = jnp.concatenate(
        [jnp.pad(edge_attr, ((0, e_pad - e), (0, 0))),
         jnp.pad(jnp.ones((e, 1), jnp.float32), ((0, e_pad - e), (0, 0))),
         jnp.zeros((e_pad, D - de - 1), jnp.float32)], axis=1)

    x_pad = jnp.pad(x, ((0, N_PAD - n), (0, 0)))
    zd = jnp.zeros((N_ACC // NUM_SUB, D), jnp.float32)

    w1a, w1b, w1e = _split_w(W1, 128, b1)
    w2a, w2b, w2e = _split_w(W2, 128, b2)
    w3a, w3b, w3e = _split_w(W3, 64, b3)

    pad_c = lambda w: jnp.pad(w, ((0, 0), (0, 128 - w.shape[1])))
    pad_r = lambda w: jnp.pad(w, ((0, 64), (0, 0)))
    w2a, w2b, w2e = pad_c(w2a), pad_c(w2b), pad_c(w2e)
    w3a, w3b = pad_r(w3a), pad_r(w3b)

    b1r = b1.reshape(1, -1)
    b2r = jnp.pad(b2, (0, 64)).reshape(1, -1)
    b3r = b3.reshape(1, -1)
    bcr = bc.reshape(1, -1)

    featscat = _make_sc_scatter(e_pad, indirect=True)
    payscat = _make_sc_scatter(e_pad, indirect=False)

    (p1,) = featscat(x_pad, src, dst, zd)
    (dsum,) = payscat(pay, dst, zd)
    h1 = _tc_layer(x_pad, p1, dsum, w1a, w1b, w1e, b1r, relu=True)
    (p2,) = featscat(h1, src, dst, zd)
    h2 = _tc_layer(h1, p2, dsum, w2a, w2b, w2e, b2r, relu=True)
    (p3,) = featscat(h2, src, dst, zd)
    out = _tc_final(h2, p3, dsum, w3a, w3b, w3e, b3r, Wc, bcr)
    return out[:n]


# speedup vs baseline: 1.0004x; 1.0004x over previous
import functools

import jax
import jax.numpy as jnp
from jax import lax
from jax.experimental import pallas as pl
from jax.experimental.pallas import tpu as pltpu
from jax.experimental.pallas import tpu_sc as plsc

N_PAD = 10240
N_ACC = 10112
CHUNK = 96
NUM_SC = 2
NUM_SUB = 16
NW = NUM_SC * NUM_SUB
D = 128
ROW_BLK = 1280


def _make_sc_scatter(e_pad, indirect):
    cpt = e_pad // (NW * CHUNK)
    rps = N_ACC // NUM_SUB
    mesh = plsc.VectorSubcoreMesh(core_axis_name="c", subcore_axis_name="s")

    NB = 4

    def body(*refs):
        if indirect:
            feat, src, dst, zd, p_out = refs[:5]
            rest = refs[5:]
            srcs, rest = rest[:NB], rest[NB:]
        else:
            feat, dst, zd, p_out = refs[:4]
            rest = refs[4:]
            srcs = (None,) * NB
            src = None
        dsts, rest = rest[:NB], rest[NB:]
        rows, rest = rest[:NB], rest[NB:]
        acc_sh, isem, gsem = rest

        c = lax.axis_index("c")
        s = lax.axis_index("s")
        wid = s * NUM_SC + c
        r0 = pl.multiple_of(s * rps, 8)
        tile_base = wid * (cpt * CHUNK)

        def hslice(ci):
            return pl.ds(pl.multiple_of(tile_base + ci * CHUNK, 8), CHUNK)

        def issue_idx(ci, b):
            pltpu.async_copy(dst.at[hslice(ci)], dsts[b], isem)
            if indirect:
                pltpu.async_copy(src.at[hslice(ci)], srcs[b], isem)

        def wait_idx(ci, b):
            pltpu.make_async_copy(dst.at[hslice(ci)], dsts[b], isem).wait()
            if indirect:
                pltpu.make_async_copy(src.at[hslice(ci)], srcs[b], isem).wait()

        def issue_gather(ci, b):
            if indirect:
                pltpu.async_copy(feat.at[srcs[b]], rows[b], gsem)
            else:
                pltpu.async_copy(feat.at[hslice(ci)], rows[b], gsem)

        def wait_gather(b):
            pltpu.make_async_copy(feat.at[pl.ds(0, CHUNK)], rows[b],
                                  gsem).wait()

        def step(ci, b, gather_next, idx_next):
            if gather_next:
                wait_idx(ci + 2, (b + 2) % NB)
                issue_gather(ci + 2, (b + 2) % NB)
            if idx_next:
                issue_idx(ci + 3, (b + 3) % NB)
            wait_gather(b)
            pltpu.sync_copy(rows[b], acc_sh.at[dsts[b]], add=True)

        pltpu.sync_copy(zd, acc_sh.at[pl.ds(r0, rps)])
        plsc.subcore_barrier()

        issue_idx(0, 0)
        issue_idx(1, 1)
        issue_idx(2, 2)
        wait_idx(0, 0)
        issue_gather(0, 0)
        wait_idx(1, 1)
        issue_gather(1, 1)
        step(0, 0, True, True)
        step(1, 1, True, True)

        nq = max(0, (cpt - 6) // NB)

        def spin(q, carry):
            for j in range(NB):
                ci = 2 + NB * q + j
                step(ci, (2 + j) % NB, True, True)
            return carry

        lax.fori_loop(0, nq, spin, 0)
        for ci in range(2 + NB * nq, cpt):
            step(ci, ci % NB, ci + 2 < cpt, ci + 3 < cpt)
        plsc.subcore_barrier()

        pltpu.sync_copy(acc_sh.at[pl.ds(r0, rps)], p_out.at[c, pl.ds(r0, rps)])

    scratch = []
    if indirect:
        scratch += [pltpu.VMEM((CHUNK,), jnp.int32)] * NB
    scratch += [pltpu.VMEM((CHUNK,), jnp.int32)] * NB
    scratch += [pltpu.VMEM((CHUNK, D), jnp.float32)] * NB
    scratch += [
        pltpu.VMEM_SHARED((N_ACC, D), jnp.float32),
        pltpu.SemaphoreType.DMA,
        pltpu.SemaphoreType.DMA,
    ]
    return pl.kernel(
        body, out_type=[jax.ShapeDtypeStruct((NUM_SC, N_PAD, D), jnp.float32)],
        mesh=mesh, scratch_types=scratch)


def _tc_layer_body(x_ref, p_ref, ds_ref, wa_ref, wb_ref, we_ref, b_ref,
                   o_ref, *, relu):
    ds = ds_ref[0] + ds_ref[1]
    deg = ds[:, 4:5] + 1.0
    sh = x_ref[...] + p_ref[0] + p_ref[1]
    acc = jnp.dot(sh, wb_ref[...], preferred_element_type=jnp.float32)
    acc += deg * jnp.dot(x_ref[...], wa_ref[...],
                         preferred_element_type=jnp.float32)
    acc += jnp.dot(ds, we_ref[...], preferred_element_type=jnp.float32)
    acc += b_ref[...]
    o_ref[...] = jnp.maximum(acc, 0.0) if relu else acc


def _tc_layer(x, p, ds, wa, wb, we, b, relu):
    din, dout = wa.shape
    grid = N_PAD // ROW_BLK
    return pl.pallas_call(
        functools.partial(_tc_layer_body, relu=relu),
        grid=(grid,),
        in_specs=[
            pl.BlockSpec((ROW_BLK, din), lambda i: (i, 0)),
            pl.BlockSpec((NUM_SC, ROW_BLK, din), lambda i: (0, i, 0)),
            pl.BlockSpec((NUM_SC, ROW_BLK, D), lambda i: (0, i, 0)),
            pl.BlockSpec((din, dout), lambda i: (0, 0)),
            pl.BlockSpec((din, dout), lambda i: (0, 0)),
            pl.BlockSpec((D, dout), lambda i: (0, 0)),
            pl.BlockSpec((1, dout), lambda i: (0, 0)),
        ],
        out_specs=pl.BlockSpec((ROW_BLK, dout), lambda i: (i, 0)),
        out_shape=jax.ShapeDtypeStruct((N_PAD, dout), jnp.float32),
    )(x, p, ds, wa, wb, we, b)


def _tc_final_body(x_ref, p_ref, ds_ref, wa_ref, wb_ref, we_ref, b_ref,
                   wc_ref, bc_ref, o_ref):
    ds = ds_ref[0] + ds_ref[1]
    deg = ds[:, 4:5] + 1.0
    sh = x_ref[...] + p_ref[0] + p_ref[1]
    h = jnp.dot(sh, wb_ref[...], preferred_element_type=jnp.float32)
    h += deg * jnp.dot(x_ref[...], wa_ref[...],
                       preferred_element_type=jnp.float32)
    h += jnp.dot(ds, we_ref[...], preferred_element_type=jnp.float32)
    h += b_ref[...]
    logits = jnp.dot(h, wc_ref[...],
                     preferred_element_type=jnp.float32) + bc_ref[...]
    m = jnp.max(logits, axis=1, keepdims=True)
    lse = m + jnp.log(jnp.sum(jnp.exp(logits - m), axis=1, keepdims=True))
    o_ref[...] = logits - lse


def _tc_final(x, p, ds, wa, wb, we, b, wc, bc):
    din, dout = wa.shape
    nc = wc.shape[1]
    grid = N_PAD // ROW_BLK
    return pl.pallas_call(
        _tc_final_body,
        grid=(grid,),
        in_specs=[
            pl.BlockSpec((ROW_BLK, din), lambda i: (i, 0)),
            pl.BlockSpec((NUM_SC, ROW_BLK, din), lambda i: (0, i, 0)),
            pl.BlockSpec((NUM_SC, ROW_BLK, D), lambda i: (0, i, 0)),
            pl.BlockSpec((din, dout), lambda i: (0, 0)),
            pl.BlockSpec((din, dout), lambda i: (0, 0)),
            pl.BlockSpec((D, dout), lambda i: (0, 0)),
            pl.BlockSpec((1, dout), lambda i: (0, 0)),
            pl.BlockSpec((dout, nc), lambda i: (0, 0)),
            pl.BlockSpec((1, nc), lambda i: (0, 0)),
        ],
        out_specs=pl.BlockSpec((ROW_BLK, nc), lambda i: (i, 0)),
        out_shape=jax.ShapeDtypeStruct((N_PAD, nc), jnp.float32),
    )(x, p, ds, wa, wb, we, b, wc, bc)


def _split_w(w, din, b):
    de = w.shape[0] - 2 * din
    wa = w[:din]
    wb = w[din:2 * din]
    we = jnp.zeros((D, w.shape[1]), jnp.float32)
    we = we.at[:de].set(w[2 * din:])
    we = we.at[4].set(b)
    return wa, wb, we


def kernel(x, edge_index, edge_attr, W1, b1, W2, b2, W3, b3, Wc, bc):
    n = x.shape[0]
    e = edge_index.shape[1]
    cpt = -(-e // (NW * CHUNK))
    if cpt % 2 == 0:
        cpt += 1
    e_pad = cpt * NW * CHUNK

    src = jnp.pad(edge_index[0].astype(jnp.int32), (0, e_pad - e))
    dst = jnp.pad(edge_index[1].astype(jnp.int32), (0, e_pad - e),
                  constant_values=n)
    de = edge_attr.shape[1]
    pay = jnp.concatenate(
        [jnp.pad(edge_attr, ((0, e_pad - e), (0, 0))),
         jnp.pad(jnp.ones((e, 1), jnp.float32), ((0, e_pad - e), (0, 0))),
         jnp.zeros((e_pad, D - de - 1), jnp.float32)], axis=1)

    x_pad = jnp.pad(x, ((0, N_PAD - n), (0, 0)))
    zd = jnp.zeros((N_ACC // NUM_SUB, D), jnp.float32)

    w1a, w1b, w1e = _split_w(W1, 128, b1)
    w2a, w2b, w2e = _split_w(W2, 128, b2)
    w3a, w3b, w3e = _split_w(W3, 64, b3)

    pad_c = lambda w: jnp.pad(w, ((0, 0), (0, 128 - w.shape[1])))
    pad_r = lambda w: jnp.pad(w, ((0, 64), (0, 0)))
    w2a, w2b, w2e = pad_c(w2a), pad_c(w2b), pad_c(w2e)
    w3a, w3b = pad_r(w3a), pad_r(w3b)

    b1r = b1.reshape(1, -1)
    b2r = jnp.pad(b2, (0, 64)).reshape(1, -1)
    b3r = b3.reshape(1, -1)
    bcr = bc.reshape(1, -1)

    featscat = _make_sc_scatter(e_pad, indirect=True)
    payscat = _make_sc_scatter(e_pad, indirect=False)

    (p1,) = featscat(x_pad, src, dst, zd)
    (dsum,) = payscat(pay, dst, zd)
    h1 = _tc_layer(x_pad, p1, dsum, w1a, w1b, w1e, b1r, relu=True)
    (p2,) = featscat(h1, src, dst, zd)
    h2 = _tc_layer(h1, p2, dsum, w2a, w2b, w2e, b2r, relu=True)
    (p3,) = featscat(h2, src, dst, zd)
    out = _tc_final(h2, p3, dsum, w3a, w3b, w3e, b3r, Wc, bcr)
    return out[:n]
